# baseline (speedup 1.0000x reference)
import jax
import jax.numpy as jnp
from jax.experimental import pallas as pl
from jax.experimental.pallas import tpu as pltpu

B = 32
MD = 1024
N = 32768
K = 4
BN = 2048
NB = N // BN


def _mp_kernel(x_ref, w_hbm,
               resid_out, xhat_out,
               wbuf, resid, babs, bval, bidx,
               idx_smem, slabs,
               sem_w, sem_idx, sem_cols):
    t = pl.program_id(0)
    n = pl.program_id(1)
    sidx = t * NB + n
    buf = jax.lax.rem(sidx, 2)

    DSPLIT = 4
    RC = MD // DSPLIT

    def w_copies(block, b):
        return [pltpu.make_async_copy(
            w_hbm.at[pl.ds(c * RC, RC), pl.ds(block * BN, BN)],
            wbuf.at[b, pl.ds(c * RC, RC)], sem_w.at[b, c])
            for c in range(DSPLIT)]

    def w_start(block, b):
        for cp in w_copies(block, b):
            cp.start()

    def w_wait(block, b):
        for cp in w_copies(block, b):
            cp.wait()

    @pl.when(sidx == 0)
    def _():
        w_start(0, 0)
        w_start(1, 1)
        resid[...] = x_ref[...]

    nsidx = sidx + 1
    @pl.when((sidx > 0) & (nsidx < K * NB))
    def _():
        nblk = jax.lax.rem(nsidx, NB)
        w_start(nblk, 1 - buf)

    @pl.when(n == 0)
    def _():
        babs[...] = jnp.full((1, B), -1.0, jnp.float32)
        bval[...] = jnp.zeros((1, B), jnp.float32)
        bidx[...] = jnp.full((1, B), N, jnp.int32)

    w_wait(n, buf)

    s = jax.lax.dot_general(wbuf[buf], resid[...],
                            (((0,), (1,)), ((), ())),
                            preferred_element_type=jnp.float32)
    a = jnp.abs(s)
    bmax = jnp.max(a, axis=0, keepdims=True)
    iota = jax.lax.broadcasted_iota(jnp.int32, (BN, B), 0)
    bloc = jnp.min(jnp.where(a == bmax, iota, N), axis=0, keepdims=True)
    bsv = jnp.sum(jnp.where(iota == bloc, s, 0.0), axis=0, keepdims=True)
    gidx = n * BN + bloc

    better = (bmax > babs[...]) | ((bmax == babs[...]) & (gidx < bidx[...]))
    babs[...] = jnp.where(better, bmax, babs[...])
    bval[...] = jnp.where(better, bsv, bval[...])
    bidx[...] = jnp.where(better, gidx, bidx[...])

    @pl.when(n == NB - 1)
    def _():
        idx_copy = pltpu.make_async_copy(bidx, idx_smem, sem_idx)
        idx_copy.start()
        idx_copy.wait()
        for r in range(B):
            base = (idx_smem[0, r] // 128) * 128
            pltpu.make_async_copy(
                w_hbm.at[:, pl.ds(base, 128)],
                slabs.at[:, pl.ds(r * 128, 128)], sem_cols.at[r]).start()
        for r in range(B):
            base = (idx_smem[0, r] // 128) * 128
            pltpu.make_async_copy(
                w_hbm.at[:, pl.ds(base, 128)],
                slabs.at[:, pl.ds(r * 128, 128)], sem_cols.at[r]).wait()
        c_iota = jax.lax.broadcasted_iota(jnp.int32, (B, B * 128), 1)
        r_iota = jax.lax.broadcasted_iota(jnp.int32, (B, B * 128), 0)
        bidx_c = jnp.transpose(bidx[...])
        valb_c = jnp.transpose(bval[...])
        sel = ((c_iota // 128) == r_iota) & \
              ((c_iota % 128) == (bidx_c % 128))
        selval = jnp.where(sel, jnp.broadcast_to(valb_c, (B, B * 128)), 0.0)
        delta = jax.lax.dot_general(selval, slabs[...],
                                    (((1,), (1,)), ((), ())),
                                    preferred_element_type=jnp.float32)
        resid[...] = resid[...] - delta

        @pl.when(t == K - 1)
        def _():
            resid_out[...] = resid[...]
            xhat_out[...] = x_ref[...] - resid[...]


def kernel(x, x_m, M, W, L, k):
    del x_m, M, L, k
    resid, xhat = pl.pallas_call(
        _mp_kernel,
        grid=(K, NB),
        in_specs=[
            pl.BlockSpec((B, MD), lambda t, n: (0, 0)),
            pl.BlockSpec(memory_space=pl.ANY),
        ],
        out_specs=[
            pl.BlockSpec((B, MD), lambda t, n: (0, 0)),
            pl.BlockSpec((B, MD), lambda t, n: (0, 0)),
        ],
        out_shape=[
            jax.ShapeDtypeStruct((B, MD), jnp.float32),
            jax.ShapeDtypeStruct((B, MD), jnp.float32),
        ],
        scratch_shapes=[
            pltpu.VMEM((2, MD, BN), jnp.float32),
            pltpu.VMEM((B, MD), jnp.float32),
            pltpu.VMEM((1, B), jnp.float32),
            pltpu.VMEM((1, B), jnp.float32),
            pltpu.VMEM((1, B), jnp.int32),
            pltpu.SMEM((1, B), jnp.int32),
            pltpu.VMEM((MD, B * 128), jnp.float32),
            pltpu.SemaphoreType.DMA((2, 4)),
            pltpu.SemaphoreType.DMA,
            pltpu.SemaphoreType.DMA((B,)),
        ],
        compiler_params=pltpu.CompilerParams(
            dimension_semantics=("arbitrary", "arbitrary"),
        ),
    )(x, W)
    return (resid, xhat)


# speedup vs baseline: 1.1237x; 1.1237x over previous
import jax
import jax.numpy as jnp
from jax.experimental import pallas as pl
from jax.experimental.pallas import tpu as pltpu

B = 32
MD = 1024
N = 32768
K = 4
BN = 2048
NB = N // BN
NBUF = 4


def _mp_kernel(x_ref, w_hbm,
               resid_out, xhat_out,
               wbuf, resid, babs, bval, bidx,
               idx_smem, slabs,
               sem_w, sem_idx, sem_cols):
    t = pl.program_id(0)
    n = pl.program_id(1)
    sidx = t * NB + n
    buf = jax.lax.rem(sidx, NBUF)

    DSPLIT = 4
    RC = MD // DSPLIT

    def w_copies(block, b):
        return [pltpu.make_async_copy(
            w_hbm.at[pl.ds(c * RC, RC), pl.ds(block * BN, BN)],
            wbuf.at[b, pl.ds(c * RC, RC)], sem_w.at[b, c])
            for c in range(DSPLIT)]

    def w_start(block, b):
        for cp in w_copies(block, b):
            cp.start()

    def w_wait(block, b):
        for cp in w_copies(block, b):
            cp.wait()

    @pl.when(sidx == 0)
    def _():
        for j in range(NBUF):
            w_start(j, j)
        resid[...] = x_ref[...]

    nsidx = sidx + NBUF - 1
    @pl.when((sidx > 0) & (nsidx < K * NB))
    def _():
        nblk = jax.lax.rem(nsidx, NB)
        w_start(nblk, jax.lax.rem(nsidx, NBUF))

    @pl.when(n == 0)
    def _():
        babs[...] = jnp.full((1, B), -1.0, jnp.float32)
        bval[...] = jnp.zeros((1, B), jnp.float32)
        bidx[...] = jnp.full((1, B), N, jnp.int32)

    w_wait(n, buf)

    s = jax.lax.dot_general(wbuf[buf], resid[...],
                            (((0,), (1,)), ((), ())),
                            preferred_element_type=jnp.float32)
    a = jnp.abs(s)
    bmax = jnp.max(a, axis=0, keepdims=True)
    iota = jax.lax.broadcasted_iota(jnp.int32, (BN, B), 0)
    bloc = jnp.min(jnp.where(a == bmax, iota, N), axis=0, keepdims=True)
    bsv = jnp.sum(jnp.where(iota == bloc, s, 0.0), axis=0, keepdims=True)
    gidx = n * BN + bloc

    better = (bmax > babs[...]) | ((bmax == babs[...]) & (gidx < bidx[...]))
    babs[...] = jnp.where(better, bmax, babs[...])
    bval[...] = jnp.where(better, bsv, bval[...])
    bidx[...] = jnp.where(better, gidx, bidx[...])

    @pl.when(n == NB - 1)
    def _():
        idx_copy = pltpu.make_async_copy(bidx, idx_smem, sem_idx)
        idx_copy.start()
        idx_copy.wait()
        for r in range(B):
            base = (idx_smem[0, r] // 128) * 128
            pltpu.make_async_copy(
                w_hbm.at[:, pl.ds(base, 128)],
                slabs.at[:, pl.ds(r * 128, 128)], sem_cols.at[r]).start()
        for r in range(B):
            base = (idx_smem[0, r] // 128) * 128
            pltpu.make_async_copy(
                w_hbm.at[:, pl.ds(base, 128)],
                slabs.at[:, pl.ds(r * 128, 128)], sem_cols.at[r]).wait()
        c_iota = jax.lax.broadcasted_iota(jnp.int32, (B, B * 128), 1)
        r_iota = jax.lax.broadcasted_iota(jnp.int32, (B, B * 128), 0)
        bidx_c = jnp.transpose(bidx[...])
        valb_c = jnp.transpose(bval[...])
        sel = ((c_iota // 128) == r_iota) & \
              ((c_iota % 128) == (bidx_c % 128))
        selval = jnp.where(sel, jnp.broadcast_to(valb_c, (B, B * 128)), 0.0)
        delta = jax.lax.dot_general(selval, slabs[...],
                                    (((1,), (1,)), ((), ())),
                                    preferred_element_type=jnp.float32)
        resid[...] = resid[...] - delta

        @pl.when(t == K - 1)
        def _():
            resid_out[...] = resid[...]
            xhat_out[...] = x_ref[...] - resid[...]


def kernel(x, x_m, M, W, L, k):
    del x_m, M, L, k
    resid, xhat = pl.pallas_call(
        _mp_kernel,
        grid=(K, NB),
        in_specs=[
            pl.BlockSpec((B, MD), lambda t, n: (0, 0)),
            pl.BlockSpec(memory_space=pl.ANY),
        ],
        out_specs=[
            pl.BlockSpec((B, MD), lambda t, n: (0, 0)),
            pl.BlockSpec((B, MD), lambda t, n: (0, 0)),
        ],
        out_shape=[
            jax.ShapeDtypeStruct((B, MD), jnp.float32),
            jax.ShapeDtypeStruct((B, MD), jnp.float32),
        ],
        scratch_shapes=[
            pltpu.VMEM((NBUF, MD, BN), jnp.float32),
            pltpu.VMEM((B, MD), jnp.float32),
            pltpu.VMEM((1, B), jnp.float32),
            pltpu.VMEM((1, B), jnp.float32),
            pltpu.VMEM((1, B), jnp.int32),
            pltpu.SMEM((1, B), jnp.int32),
            pltpu.VMEM((MD, B * 128), jnp.float32),
            pltpu.SemaphoreType.DMA((NBUF, 4)),
            pltpu.SemaphoreType.DMA,
            pltpu.SemaphoreType.DMA((B,)),
        ],
        compiler_params=pltpu.CompilerParams(
            dimension_semantics=("arbitrary", "arbitrary"),
        ),
    )(x, W)
    return (resid, xhat)


# speedup vs baseline: 1.1669x; 1.0384x over previous
import jax
import jax.numpy as jnp
from jax.experimental import pallas as pl
from jax.experimental.pallas import tpu as pltpu

B = 32
MD = 1024
N = 32768
K = 4
BN = 1024
NB = N // BN
NBUF = 4
RES = 15
NS = NB - RES
TOT_STREAM = NB + (K - 1) * NS
RG = 8
NR = B // RG
DSPLIT = 4
RC = MD // DSPLIT


def _scan_block(s, bid, babs, bval, bidx):
    a = jnp.abs(s)
    bmax = jnp.max(a, axis=0, keepdims=True)
    iota = jax.lax.broadcasted_iota(jnp.int32, (BN, B), 0)
    bloc = jnp.min(jnp.where(a == bmax, iota, N), axis=0, keepdims=True)
    bsv = jnp.sum(jnp.where(iota == bloc, s, 0.0), axis=0, keepdims=True)
    gidx = bid * BN + bloc
    better = (bmax > babs[...]) | ((bmax == babs[...]) & (gidx < bidx[...]))
    babs[...] = jnp.where(better, bmax, babs[...])
    bval[...] = jnp.where(better, bsv, bval[...])
    bidx[...] = jnp.where(better, gidx, bidx[...])


def _mp_kernel(x_ref, w_hbm,
               resid_out, xhat_out,
               wbuf, wres, sbuf, resid, resid_b, babs, bval, bidx,
               idx_smem,
               sem_w, sem_idx, sem_g):
    t = pl.program_id(0)
    n = pl.program_id(1)

    is_t0 = t == 0
    r_cnt = jnp.minimum((n + 1) // 2, RES)
    is_res = (~is_t0) & (jax.lax.rem(n, 2) == 0) & (n // 2 < RES)
    s_local = n - r_cnt
    bid = jnp.where(is_t0, n,
                    jnp.where(is_res, n // 2, RES + s_local))
    sidx = jnp.where(is_t0, n, NB + (t - 1) * NS + s_local)
    buf = jax.lax.rem(sidx, NBUF)

    def c2bid(c):
        return jnp.where(c < NB, c, RES + jax.lax.rem(c - NB, NS))

    def w_start(c):
        blk = c2bid(c)
        b = jax.lax.rem(c, NBUF)
        for j in range(DSPLIT):
            pltpu.make_async_copy(
                w_hbm.at[pl.ds(j * RC, RC), pl.ds(blk * BN, BN)],
                wbuf.at[b, pl.ds(j * RC, RC)], sem_w.at[b, j]).start()

    @pl.when((t == 0) & (n == 0))
    def _():
        for j in range(NBUF):
            w_start(j)
        resid[...] = x_ref[...]
        resid_b[...] = x_ref[...].astype(jnp.bfloat16)

    nxt = sidx + NBUF - 1
    @pl.when((~is_res) & (sidx > 0) & (nxt < TOT_STREAM))
    def _():
        w_start(nxt)

    @pl.when(n == 0)
    def _():
        babs[...] = jnp.full((1, B), -1.0, jnp.float32)
        bval[...] = jnp.zeros((1, B), jnp.float32)
        bidx[...] = jnp.full((1, B), N, jnp.int32)

    @pl.when(~is_res)
    def _():
        for j in range(DSPLIT):
            pltpu.make_async_copy(
                w_hbm.at[pl.ds(j * RC, RC), pl.ds(0, BN)],
                wbuf.at[buf, pl.ds(j * RC, RC)], sem_w.at[buf, j]).wait()

        @pl.when(is_t0 & (bid < RES))
        def _():
            wres[bid] = wbuf[buf].astype(jnp.bfloat16)

        s = jax.lax.dot_general(wbuf[buf], resid[...],
                                (((0,), (1,)), ((), ())),
                                preferred_element_type=jnp.float32)
        _scan_block(s, bid, babs, bval, bidx)

    @pl.when(is_res)
    def _():
        s = jax.lax.dot_general(wres[bid], resid_b[...],
                                (((0,), (1,)), ((), ())),
                                preferred_element_type=jnp.float32)
        _scan_block(s, bid, babs, bval, bidx)

    @pl.when(n == NB - 1)
    def _():
        idx_copy = pltpu.make_async_copy(bidx, idx_smem, sem_idx)
        idx_copy.start()
        idx_copy.wait()

        def g_start(j):
            for i in range(RG):
                base = (idx_smem[0, j * RG + i] // 128) * 128
                pltpu.make_async_copy(
                    w_hbm.at[:, pl.ds(base, 128)],
                    sbuf.at[j % 2, :, pl.ds(i * 128, 128)],
                    sem_g.at[j % 2, i]).start()

        def g_wait(j):
            for i in range(RG):
                base = (idx_smem[0, j * RG + i] // 128) * 128
                pltpu.make_async_copy(
                    w_hbm.at[:, pl.ds(base, 128)],
                    sbuf.at[j % 2, :, pl.ds(i * 128, 128)],
                    sem_g.at[j % 2, i]).wait()

        g_start(0)
        g_start(1)
        c_iota = jax.lax.broadcasted_iota(jnp.int32, (B, RG * 128), 1)
        r_iota = jax.lax.broadcasted_iota(jnp.int32, (B, RG * 128), 0)
        bidx_c = jnp.transpose(bidx[...])
        bval_c = jnp.transpose(bval[...])
        dtot = jnp.zeros((B, MD), jnp.float32)
        for j in range(NR):
            g_wait(j)
            if j + 2 < NR:
                g_start(j + 2)
            sel = ((c_iota // 128) == (r_iota - j * RG)) & \
                  ((c_iota % 128) == (bidx_c % 128))
            selval = jnp.where(sel, jnp.broadcast_to(bval_c, (B, RG * 128)),
                               0.0)
            dtot = dtot + jax.lax.dot_general(
                selval, sbuf[j % 2],
                (((1,), (1,)), ((), ())),
                preferred_element_type=jnp.float32)
        resid[...] = resid[...] - dtot
        resid_b[...] = resid[...].astype(jnp.bfloat16)

        @pl.when(t == K - 1)
        def _():
            resid_out[...] = resid[...]
            xhat_out[...] = x_ref[...] - resid[...]


def kernel(x, x_m, M, W, L, k):
    del x_m, M, L, k
    resid, xhat = pl.pallas_call(
        _mp_kernel,
        grid=(K, NB),
        in_specs=[
            pl.BlockSpec((B, MD), lambda t, n: (0, 0)),
            pl.BlockSpec(memory_space=pl.ANY),
        ],
        out_specs=[
            pl.BlockSpec((B, MD), lambda t, n: (0, 0)),
            pl.BlockSpec((B, MD), lambda t, n: (0, 0)),
        ],
        out_shape=[
            jax.ShapeDtypeStruct((B, MD), jnp.float32),
            jax.ShapeDtypeStruct((B, MD), jnp.float32),
        ],
        scratch_shapes=[
            pltpu.VMEM((NBUF, MD, BN), jnp.float32),
            pltpu.VMEM((RES, MD, BN), jnp.bfloat16),
            pltpu.VMEM((2, MD, RG * 128), jnp.float32),
            pltpu.VMEM((B, MD), jnp.float32),
            pltpu.VMEM((B, MD), jnp.bfloat16),
            pltpu.VMEM((1, B), jnp.float32),
            pltpu.VMEM((1, B), jnp.float32),
            pltpu.VMEM((1, B), jnp.int32),
            pltpu.SMEM((1, B), jnp.int32),
            pltpu.SemaphoreType.DMA((NBUF, DSPLIT)),
            pltpu.SemaphoreType.DMA,
            pltpu.SemaphoreType.DMA((2, RG)),
        ],
        compiler_params=pltpu.CompilerParams(
            dimension_semantics=("arbitrary", "arbitrary"),
        ),
    )(x, W)
    return (resid, xhat)


# speedup vs baseline: 1.2121x; 1.0388x over previous
import jax
import jax.numpy as jnp
from jax.experimental import pallas as pl
from jax.experimental.pallas import tpu as pltpu

B = 32
MD = 1024
N = 32768
K = 4
BN = 1024
NB = N // BN
NBUF = 4
RES = 15
NS = NB - RES
TOT_STREAM = NB + (K - 1) * NS
SUB = 128
NSUB = BN // SUB
RG = 4
NR = B // RG
DSPLIT = 4
RC = MD // DSPLIT


def _scan_block(s, bid, babs, bm_all):
    a = jnp.abs(s)
    bm = jnp.concatenate(
        [jnp.max(a[i * SUB:(i + 1) * SUB, :], axis=0, keepdims=True)
         for i in range(NSUB)], axis=0)
    babs[...] = jnp.maximum(babs[...],
                            jnp.max(bm, axis=0, keepdims=True))
    bm_all[pl.ds(bid * NSUB, NSUB), :] = bm


def _mp_kernel(x_ref, w_hbm,
               resid_out, xhat_out,
               wbuf, wres, sbuf, bm_all, resid, resid_b, babs, bval, bidx,
               idx_smem,
               sem_w, sem_idx, sem_g):
    t = pl.program_id(0)
    n = pl.program_id(1)

    is_t0 = t == 0
    r_cnt = jnp.minimum((n + 1) // 2, RES)
    is_res = (~is_t0) & (jax.lax.rem(n, 2) == 0) & (n // 2 < RES)
    s_local = n - r_cnt
    bid = jnp.where(is_t0, n,
                    jnp.where(is_res, n // 2, RES + s_local))
    sidx = jnp.where(is_t0, n, NB + (t - 1) * NS + s_local)
    buf = jax.lax.rem(sidx, NBUF)

    def w_start(c):
        blk = jnp.where(c < NB, c, RES + jax.lax.rem(c - NB, NS))
        b = jax.lax.rem(c, NBUF)
        for j in range(DSPLIT):
            pltpu.make_async_copy(
                w_hbm.at[pl.ds(j * RC, RC), pl.ds(blk * BN, BN)],
                wbuf.at[b, pl.ds(j * RC, RC)], sem_w.at[b, j]).start()

    @pl.when((t == 0) & (n == 0))
    def _():
        for j in range(NBUF):
            w_start(j)
        resid[...] = x_ref[...]
        resid_b[...] = x_ref[...].astype(jnp.bfloat16)

    nxt = sidx + NBUF - 1
    @pl.when((~is_res) & (sidx > 0) & (nxt < TOT_STREAM))
    def _():
        w_start(nxt)

    @pl.when(n == 0)
    def _():
        babs[...] = jnp.full((1, B), -1.0, jnp.float32)

    @pl.when(~is_res)
    def _():
        for j in range(DSPLIT):
            pltpu.make_async_copy(
                w_hbm.at[pl.ds(j * RC, RC), pl.ds(0, BN)],
                wbuf.at[buf, pl.ds(j * RC, RC)], sem_w.at[buf, j]).wait()

        @pl.when(is_t0 & (bid < RES))
        def _():
            wres[bid] = wbuf[buf].astype(jnp.bfloat16)

        s = jax.lax.dot_general(wbuf[buf], resid[...],
                                (((0,), (1,)), ((), ())),
                                preferred_element_type=jnp.float32)
        _scan_block(s, bid, babs, bm_all)

    @pl.when(is_res)
    def _():
        s = jax.lax.dot_general(wres[bid], resid_b[...],
                                (((0,), (1,)), ((), ())),
                                preferred_element_type=jnp.float32)
        _scan_block(s, bid, babs, bm_all)

    @pl.when(n == NB - 1)
    def _():
        io_s = jax.lax.broadcasted_iota(jnp.int32, (NB * NSUB, B), 0)
        wc = jnp.min(jnp.where(bm_all[...] == babs[...], io_s, NB * NSUB),
                     axis=0, keepdims=True)
        bidx[...] = wc
        idx_copy = pltpu.make_async_copy(bidx, idx_smem, sem_idx)
        idx_copy.start()
        idx_copy.wait()

        def g_start(j):
            for i in range(RG):
                base = idx_smem[0, j * RG + i] * SUB
                pltpu.make_async_copy(
                    w_hbm.at[:, pl.ds(base, SUB)],
                    sbuf.at[j % 2, :, pl.ds(i * SUB, SUB)],
                    sem_g.at[j % 2, i]).start()

        def g_wait(j):
            for i in range(RG):
                pltpu.make_async_copy(
                    w_hbm.at[:, pl.ds(0, SUB)],
                    sbuf.at[j % 2, :, pl.ds(i * SUB, SUB)],
                    sem_g.at[j % 2, i]).wait()

        g_start(0)
        g_start(1)
        GW = RG * SUB
        io = jax.lax.broadcasted_iota(jnp.int32, (GW, B), 0)
        rl = jax.lax.broadcasted_iota(jnp.int32, (GW, B), 1)
        c_iota = jax.lax.broadcasted_iota(jnp.int32, (B, GW), 1)
        r_iota = jax.lax.broadcasted_iota(jnp.int32, (B, GW), 0)
        lanes = jax.lax.broadcasted_iota(jnp.int32, (1, B), 1)
        dtot = jnp.zeros((B, MD), jnp.float32)
        for j in range(NR):
            g_wait(j)
            ss = jax.lax.dot_general(sbuf[j % 2], resid[...],
                                     (((0,), (1,)), ((), ())),
                                     preferred_element_type=jnp.float32)
            cond = (io // SUB) == (rl - j * RG)
            aa = jnp.abs(ss)
            am = jnp.max(jnp.where(cond, aa, -1.0), axis=0, keepdims=True)
            lloc = jnp.min(jnp.where(cond & (aa == am), io, GW),
                           axis=0, keepdims=True)
            sv = jnp.sum(jnp.where(io == lloc, ss, 0.0),
                         axis=0, keepdims=True)
            inrng = (lanes >= j * RG) & (lanes < (j + 1) * RG)
            bidx[...] = jnp.where(inrng, wc * SUB + jax.lax.rem(lloc, SUB),
                                  bidx[...])
            bval[...] = jnp.where(inrng, sv, bval[...])
            ll_c = jnp.transpose(lloc)
            sv_c = jnp.transpose(sv)
            sel = ((c_iota // SUB) == (r_iota - j * RG)) & \
                  ((c_iota % SUB) == jax.lax.rem(ll_c, SUB))
            selval = jnp.where(sel, jnp.broadcast_to(sv_c, (B, GW)), 0.0)
            dtot = dtot + jax.lax.dot_general(
                selval, sbuf[j % 2],
                (((1,), (1,)), ((), ())),
                preferred_element_type=jnp.float32)
            if j + 2 < NR:
                g_start(j + 2)
        resid[...] = resid[...] - dtot
        resid_b[...] = resid[...].astype(jnp.bfloat16)

        @pl.when(t == K - 1)
        def _():
            resid_out[...] = resid[...]
            xhat_out[...] = x_ref[...] - resid[...]


def kernel(x, x_m, M, W, L, k):
    del x_m, M, L, k
    resid, xhat = pl.pallas_call(
        _mp_kernel,
        grid=(K, NB),
        in_specs=[
            pl.BlockSpec((B, MD), lambda t, n: (0, 0)),
            pl.BlockSpec(memory_space=pl.ANY),
        ],
        out_specs=[
            pl.BlockSpec((B, MD), lambda t, n: (0, 0)),
            pl.BlockSpec((B, MD), lambda t, n: (0, 0)),
        ],
        out_shape=[
            jax.ShapeDtypeStruct((B, MD), jnp.float32),
            jax.ShapeDtypeStruct((B, MD), jnp.float32),
        ],
        scratch_shapes=[
            pltpu.VMEM((NBUF, MD, BN), jnp.float32),
            pltpu.VMEM((RES, MD, BN), jnp.bfloat16),
            pltpu.VMEM((2, MD, RG * SUB), jnp.float32),
            pltpu.VMEM((NB * NSUB, B), jnp.float32),
            pltpu.VMEM((B, MD), jnp.float32),
            pltpu.VMEM((B, MD), jnp.bfloat16),
            pltpu.VMEM((1, B), jnp.float32),
            pltpu.VMEM((1, B), jnp.float32),
            pltpu.VMEM((1, B), jnp.int32),
            pltpu.SMEM((1, B), jnp.int32),
            pltpu.SemaphoreType.DMA((NBUF, DSPLIT)),
            pltpu.SemaphoreType.DMA,
            pltpu.SemaphoreType.DMA((2, RG)),
        ],
        compiler_params=pltpu.CompilerParams(
            dimension_semantics=("arbitrary", "arbitrary"),
        ),
    )(x, W)
    return (resid, xhat)
